# Initial kernel scaffold; baseline (speedup 1.0000x reference)
import jax
import jax.numpy as jnp
from jax.experimental import pallas as pl


def kernel(x, w_ih_0_0, w_hh_0_0, b_ih_0_0, b_hh_0_0, w_ih_0_1, w_hh_0_1, b_ih_0_1, b_hh_0_1, w_ih_1_0, w_hh_1_0, b_ih_1_0, b_hh_1_0, w_ih_1_1, w_hh_1_1, b_ih_1_1, b_hh_1_1):
    raise NotImplementedError("write your pallas kernel here")


# speedup vs baseline: 1.2527x; 1.2527x over previous
import jax
import jax.numpy as jnp
from jax.experimental import pallas as pl
from jax.experimental.pallas import tpu as pltpu


def _lstm_gates(gates, c, H):
    sig = jax.nn.sigmoid(gates[:, :3 * H])
    i_g = sig[:, 0:H]
    f_g = sig[:, H:2 * H]
    o_g = sig[:, 2 * H:3 * H]
    g_g = jnp.tanh(gates[:, 3 * H:])
    c_new = f_g * c + i_g * g_g
    h_new = o_g * jnp.tanh(c_new)
    return h_new, c_new


def _layer0_kernel(x_ref, wih_ref, whh_ref, b_ref, out_ref, h_ref, c_ref):
    d = pl.program_id(0)
    k = pl.program_id(1)

    @pl.when(k == 0)
    def _():
        h_ref[...] = jnp.zeros_like(h_ref)
        c_ref[...] = jnp.zeros_like(c_ref)

    Tc = x_ref.shape[0]
    H = h_ref.shape[1]
    wih = wih_ref[0]
    whh = whh_ref[0]
    bias = b_ref[0]

    def body(j, carry):
        h, c = carry
        row = jnp.where(d == 0, j, Tc - 1 - j)
        gates = (jnp.dot(x_ref[row], wih, preferred_element_type=jnp.float32)
                 + jnp.dot(h, whh, preferred_element_type=jnp.float32)
                 + bias)
        h, c = _lstm_gates(gates, c, H)
        out_ref[row] = h
        return h, c

    carry = jax.lax.fori_loop(0, Tc, body, (h_ref[...], c_ref[...]),
                              unroll=min(Tc, 8))
    h_ref[...], c_ref[...] = carry


def _layer1_kernel(xf_ref, xb_ref, wih_ref, whh_ref, b_ref, out_ref,
                   h_ref, c_ref):
    d = pl.program_id(0)
    k = pl.program_id(1)

    @pl.when(k == 0)
    def _():
        h_ref[...] = jnp.zeros_like(h_ref)
        c_ref[...] = jnp.zeros_like(c_ref)

    Tc = xf_ref.shape[0]
    H = h_ref.shape[1]
    wih = wih_ref[0]
    whh = whh_ref[0]
    bias = b_ref[0]

    def body(j, carry):
        h, c = carry
        row = jnp.where(d == 0, j, Tc - 1 - j)
        gates = (jnp.dot(xf_ref[row], wih[:H], preferred_element_type=jnp.float32)
                 + jnp.dot(xb_ref[row], wih[H:], preferred_element_type=jnp.float32)
                 + jnp.dot(h, whh, preferred_element_type=jnp.float32)
                 + bias)
        h, c = _lstm_gates(gates, c, H)
        out_ref[row] = h
        return h, c

    carry = jax.lax.fori_loop(0, Tc, body, (h_ref[...], c_ref[...]),
                              unroll=min(Tc, 8))
    h_ref[...], c_ref[...] = carry


_VMEM_LIMIT = 100 * 1024 * 1024


def _pick_tc(T, rows_bytes_per_t, budget):
    best = 1
    for c in range(1, T + 1):
        if T % c == 0 and c * rows_bytes_per_t * 2 <= budget:
            best = c
    return best


def _chunk_index(nchunks):
    def c(d, k):
        return k + d * (nchunks - 1 - 2 * k)
    return c


def _run_layer0(x_tm, wih, whh, bias, T, Bp, H):
    D = x_tm.shape[-1]
    Tc = _pick_tc(T, Bp * (D + 2 * H) * 4, 48 * 1024 * 1024)
    nchunks = T // Tc
    c = _chunk_index(nchunks)
    return pl.pallas_call(
        _layer0_kernel,
        out_shape=jax.ShapeDtypeStruct((T, Bp, 2 * H), jnp.float32),
        grid_spec=pltpu.PrefetchScalarGridSpec(
            num_scalar_prefetch=0,
            grid=(2, nchunks),
            in_specs=[
                pl.BlockSpec((Tc, Bp, D), lambda d, k: (c(d, k), 0, 0)),
                pl.BlockSpec((1, D, 4 * H), lambda d, k: (d, 0, 0),
                             pipeline_mode=pl.Buffered(1)),
                pl.BlockSpec((1, H, 4 * H), lambda d, k: (d, 0, 0),
                             pipeline_mode=pl.Buffered(1)),
                pl.BlockSpec((1, 1, 4 * H), lambda d, k: (d, 0, 0),
                             pipeline_mode=pl.Buffered(1)),
            ],
            out_specs=pl.BlockSpec((Tc, Bp, H), lambda d, k: (c(d, k), 0, d)),
            scratch_shapes=[
                pltpu.VMEM((Bp, H), jnp.float32),
                pltpu.VMEM((Bp, H), jnp.float32),
            ],
        ),
        compiler_params=pltpu.CompilerParams(
            dimension_semantics=("parallel", "arbitrary"),
            vmem_limit_bytes=_VMEM_LIMIT),
    )(x_tm, wih, whh, bias)


def _run_layer1(prev, wih, whh, bias, T, Bp, H):
    Tc = _pick_tc(T, Bp * (2 * H + 2 * H) * 4, 48 * 1024 * 1024)
    nchunks = T // Tc
    c = _chunk_index(nchunks)
    return pl.pallas_call(
        _layer1_kernel,
        out_shape=jax.ShapeDtypeStruct((T, Bp, 2 * H), jnp.float32),
        grid_spec=pltpu.PrefetchScalarGridSpec(
            num_scalar_prefetch=0,
            grid=(2, nchunks),
            in_specs=[
                pl.BlockSpec((Tc, Bp, H), lambda d, k: (c(d, k), 0, 0)),
                pl.BlockSpec((Tc, Bp, H), lambda d, k: (c(d, k), 0, 1)),
                pl.BlockSpec((1, 2 * H, 4 * H), lambda d, k: (d, 0, 0),
                             pipeline_mode=pl.Buffered(1)),
                pl.BlockSpec((1, H, 4 * H), lambda d, k: (d, 0, 0),
                             pipeline_mode=pl.Buffered(1)),
                pl.BlockSpec((1, 1, 4 * H), lambda d, k: (d, 0, 0),
                             pipeline_mode=pl.Buffered(1)),
            ],
            out_specs=pl.BlockSpec((Tc, Bp, H), lambda d, k: (c(d, k), 0, d)),
            scratch_shapes=[
                pltpu.VMEM((Bp, H), jnp.float32),
                pltpu.VMEM((Bp, H), jnp.float32),
            ],
        ),
        compiler_params=pltpu.CompilerParams(
            dimension_semantics=("parallel", "arbitrary"),
            vmem_limit_bytes=_VMEM_LIMIT),
    )(prev, prev, wih, whh, bias)


def _repack(w, axis):
    i, f, g, o = jnp.split(w, 4, axis=axis)
    return jnp.concatenate([i, f, o, g], axis=axis)


def _prep_dir(w_ih, w_hh, b_ih, b_hh):
    wih_t = _repack(w_ih, 0).T
    whh_t = _repack(w_hh, 0).T
    bias = _repack(b_ih + b_hh, 0).reshape(1, -1)
    return wih_t, whh_t, bias


def kernel(x,
           w_ih_0_0, w_hh_0_0, b_ih_0_0, b_hh_0_0,
           w_ih_0_1, w_hh_0_1, b_ih_0_1, b_hh_0_1,
           w_ih_1_0, w_hh_1_0, b_ih_1_0, b_hh_1_0,
           w_ih_1_1, w_hh_1_1, b_ih_1_1, b_hh_1_1):
    B, T, D = x.shape
    H = w_hh_0_0.shape[1]
    Bp = ((B + 7) // 8) * 8

    x_tm = jnp.transpose(x, (1, 0, 2))
    if Bp != B:
        x_tm = jnp.pad(x_tm, ((0, 0), (0, Bp - B), (0, 0)))

    w0f = _prep_dir(w_ih_0_0, w_hh_0_0, b_ih_0_0, b_hh_0_0)
    w0b = _prep_dir(w_ih_0_1, w_hh_0_1, b_ih_0_1, b_hh_0_1)
    w1f = _prep_dir(w_ih_1_0, w_hh_1_0, b_ih_1_0, b_hh_1_0)
    w1b = _prep_dir(w_ih_1_1, w_hh_1_1, b_ih_1_1, b_hh_1_1)

    wih0 = jnp.stack([w0f[0], w0b[0]])
    whh0 = jnp.stack([w0f[1], w0b[1]])
    b0 = jnp.stack([w0f[2], w0b[2]])
    wih1 = jnp.stack([w1f[0], w1b[0]])
    whh1 = jnp.stack([w1f[1], w1b[1]])
    b1 = jnp.stack([w1f[2], w1b[2]])

    out0 = _run_layer0(x_tm, wih0, whh0, b0, T, Bp, H)
    out1 = _run_layer1(out0, wih1, whh1, b1, T, Bp, H)

    return jnp.transpose(out1[:, :B], (1, 0, 2))
